# baseline (speedup 1.0000x reference)
import jax
import jax.numpy as jnp
from jax import lax
from jax.experimental import pallas as pl
from jax.experimental.pallas import tpu as pltpu

_NB = 100
_RW = 104
_NR = 100
_N = _NR * _RW
_E = 128
_U = 256
_H = _RW
_GR = 20
_RN = _GR * _RW
_NRB = _NR // _GR
_NP = _N + 2 * _H

_BF = jnp.bfloat16


def _dot(a, b):
    return jnp.dot(a, b, preferred_element_type=jnp.float32)


def _rne(x):
    u = lax.bitcast_convert_type(x, jnp.uint32)
    r = (u + jnp.uint32(0x7FFF) + ((u >> 16) & jnp.uint32(1))) & jnp.uint32(0xFFFF0000)
    return lax.bitcast_convert_type(r, jnp.float32)


def _gnn_body(xsa_ref, in_W_ref, in_b_ref, cW1_ref, rW1_ref,
              cb1_ref, rb1_ref, W2c_ref, W2r_ref, cb2_ref,
              rb2_ref, eW1_ref, eb1_ref, eW2_ref, eb2_ref,
              ro_row_ref, robc_ref, ro2_ref, rdc_ref, out_ref, buf_ref):
    s = pl.program_id(1)
    rb = pl.program_id(2)
    base = rb * _RN
    p_in = (s + 1) % 2
    p_out = s % 2

    def write_h(h):
        buf_ref[p_out, pl.ds(base + _H, _RN), :] = h

        @pl.when(rb == 0)
        def _():
            buf_ref[p_out, pl.ds(0, _H), :] = h[0:_H]

        @pl.when(rb == _NRB - 1)
        def _():
            buf_ref[p_out, pl.ds(_N + _H, _H), :] = h[_RN - _H:_RN]

    @pl.when(s == 0)
    def _embed():
        h0 = xsa_ref[0]
        write_h(h0)
        out_ref[0] = h0

    @pl.when(s > 0)
    def _layer():
        S = buf_ref[p_in, pl.ds(base, _RN + 2 * _H), :].astype(_BF)
        self_bf = S[_H:_H + _RN]

        h_up = S[0:_RN]
        h_dn = S[2 * _H:2 * _H + _RN]
        i0 = lax.broadcasted_iota(jnp.int32, (_RN, 1), 0) % _RW
        h_lf = jnp.where(i0 == 0, self_bf, S[_H - 1:_H - 1 + _RN])
        h_rt = jnp.where(i0 == _NB - 1, self_bf, S[_H + 1:_H + 1 + _RN])

        def msg(nbr, W1, b1, W2, b2):
            x = jnp.concatenate([self_bf, nbr], axis=1)
            a = jnp.maximum(_dot(x, W1) + b1, 0.0).astype(_BF)
            return _dot(a, W2) + b2

        m_up = msg(h_up, cW1_ref[...], cb1_ref[...], W2c_ref[...],
                   cb2_ref[...])
        m_dn = msg(h_dn, cW1_ref[...], cb1_ref[...], W2c_ref[...],
                   cb2_ref[...])
        m_lf = msg(h_lf, rW1_ref[...], rb1_ref[...], W2r_ref[...],
                   rb2_ref[...])
        m_rt = msg(h_rt, rW1_ref[...], rb1_ref[...], W2r_ref[...],
                   rb2_ref[...])
        m_agg = ((m_up + m_dn) + m_lf) + m_rt

        xe = jnp.concatenate([self_bf, m_agg.astype(_BF)], axis=1)
        g = jnp.maximum(_dot(xe, eW1_ref[...]) + eb1_ref[...],
                        0.0).astype(_BF)
        hn = _dot(g, eW2_ref[...]) + eb2_ref[...]
        write_h(hn)
        out_ref[0] = hn


def _pad_rows(x2d):
    b = x2d.shape[0]
    x3 = x2d.reshape(b, _NR, _NB)
    x3 = jnp.pad(x3, ((0, 0), (0, 0), (0, _RW - _NB)))
    return x3.reshape(b, _N)


def _bfr(w):
    return w.astype(_BF)


def kernel(state, actor, up, down, left, right, in_W, in_b,
           row_W1, row_b1, row_W2, row_b2,
           col_W1, col_b1, col_W2, col_b2,
           emb_W1, emb_b1, emb_W2, emb_b2,
           ro_W, ro_b, ro2_W, ro2_b):
    B = state.shape[0]

    def _bdo(x, W):
        return jnp.dot(x.astype(_BF), W.astype(_BF),
                       precision=lax.Precision.HIGHEST,
                       preferred_element_type=jnp.float32)

    sa = jnp.concatenate([state.reshape(B, -1)[..., None],
                          actor.reshape(B, -1)[..., None]], axis=-1)
    h0 = _bdo(sa, in_W) + in_b
    h0p = jnp.pad(h0.reshape(B, _NR, _NB, _E),
                  ((0, 0), (0, 0), (0, _RW - _NB), (0, 0))).reshape(B, _N, _E)

    grid = (B, 4, _NRB)

    def full(shape):
        return pl.BlockSpec(shape, lambda b, s, r: (0,) * len(shape))

    out = pl.pallas_call(
        _gnn_body,
        grid=grid,
        in_specs=[
            pl.BlockSpec((1, _RN, _E), lambda b, s, r: (b, r, 0)),
            full((2, _E)),
            full((1, _E)),
            full((2 * _E, _U)),
            full((2 * _E, _U)),
            full((1, _U)),
            full((1, _U)),
            full((_U, _E)),
            full((_U, _E)),
            full((1, _E)),
            full((1, _E)),
            full((2 * _E, _U)),
            full((1, _U)),
            full((_U, _E)),
            full((1, _E)),
            full((1, _E)),
            full((1, 1)),
            pl.BlockSpec((_RN, 1), lambda b, s, r: (r, 0)),
            full((1, 1)),
        ],
        out_specs=pl.BlockSpec((1, _RN, _E), lambda b, s, r: (b, r, 0)),
        out_shape=jax.ShapeDtypeStruct((B, _N, _E), jnp.float32),
        scratch_shapes=[pltpu.VMEM((2, _NP, _E), jnp.float32)],
        compiler_params=pltpu.CompilerParams(
            dimension_semantics=("arbitrary", "arbitrary", "arbitrary")),
    )(h0p,
      _bfr(in_W).astype(jnp.float32),
      in_b.reshape(1, _E),
      _bfr(col_W1), _bfr(row_W1),
      col_b1.reshape(1, _U), row_b1.reshape(1, _U),
      _bfr(col_W2), _bfr(row_W2),
      col_b2.reshape(1, _E), row_b2.reshape(1, _E),
      _bfr(emb_W1),
      emb_b1.reshape(1, _U),
      _bfr(emb_W2), emb_b2.reshape(1, _E),
      _bfr(ro_W).astype(jnp.float32).reshape(1, _E),
      ro_b.reshape(1, 1),
      _bfr(_pad_rows(ro2_W.reshape(1, -1))).astype(jnp.float32).reshape(_N, 1),
      ro2_b.reshape(1, 1))
    h = out.reshape(B, _NR, _RW, _E)[:, :, :_NB, :].reshape(B, _NR * _NB, _E)
    def _bd(x, W):
        return jnp.dot(x.astype(_BF), W.astype(_BF),
                       precision=lax.Precision.HIGHEST,
                       preferred_element_type=jnp.float32)
    q = jnp.squeeze(_bd(h, ro_W) + ro_b, axis=-1)
    q = _bd(q, ro2_W) + ro2_b
    return q


# speedup vs baseline: 1.1629x; 1.1629x over previous
import jax
import jax.numpy as jnp
from jax import lax
from jax.experimental import pallas as pl
from jax.experimental.pallas import tpu as pltpu

_NB = 100
_RW = 104
_NR = 100
_N = _NR * _RW
_E = 128
_U = 256
_H = _RW
_GR = 20
_RN = _GR * _RW
_NRB = _NR // _GR
_NP = _N + 2 * _H

_BF = jnp.bfloat16


def _dot(a, b):
    return jnp.dot(a, b, preferred_element_type=jnp.float32)


def _rne(x):
    u = lax.bitcast_convert_type(x, jnp.uint32)
    r = (u + jnp.uint32(0x7FFF) + ((u >> 16) & jnp.uint32(1))) & jnp.uint32(0xFFFF0000)
    return lax.bitcast_convert_type(r, jnp.float32)


def _gnn_body(xsa_ref, in_W_ref, in_b_ref, cW1_ref, rW1_ref,
              cb1_ref, rb1_ref, W2c_ref, W2r_ref, cb2_ref,
              rb2_ref, eW1_ref, eb1_ref, eW2_ref, eb2_ref,
              ro_row_ref, robc_ref, ro2_ref, rdc_ref, out_ref, buf_ref):
    s = pl.program_id(1)
    rb = pl.program_id(2)
    base = rb * _RN
    p_in = (s + 1) % 2
    p_out = s % 2

    def write_h(h):
        buf_ref[p_out, pl.ds(base + _H, _RN), :] = h

        @pl.when(rb == 0)
        def _():
            buf_ref[p_out, pl.ds(0, _H), :] = h[0:_H]

        @pl.when(rb == _NRB - 1)
        def _():
            buf_ref[p_out, pl.ds(_N + _H, _H), :] = h[_RN - _H:_RN]

    @pl.when(s == 0)
    def _embed():
        h0 = xsa_ref[0]
        write_h(h0)

    @pl.when(s > 0)
    def _layer():
        S = buf_ref[p_in, pl.ds(base, _RN + 2 * _H), :].astype(_BF)
        self_bf = S[_H:_H + _RN]

        h_up = S[0:_RN]
        h_dn = S[2 * _H:2 * _H + _RN]
        i0 = lax.broadcasted_iota(jnp.int32, (_RN, 1), 0) % _RW
        h_lf = jnp.where(i0 == 0, self_bf, S[_H - 1:_H - 1 + _RN])
        h_rt = jnp.where(i0 == _NB - 1, self_bf, S[_H + 1:_H + 1 + _RN])

        def msg(nbr, W1, b1, W2, b2):
            x = jnp.concatenate([self_bf, nbr], axis=1)
            a = jnp.maximum(_dot(x, W1) + b1, 0.0).astype(_BF)
            return _dot(a, W2) + b2

        m_up = msg(h_up, cW1_ref[...], cb1_ref[...], W2c_ref[...],
                   cb2_ref[...])
        m_dn = msg(h_dn, cW1_ref[...], cb1_ref[...], W2c_ref[...],
                   cb2_ref[...])
        m_lf = msg(h_lf, rW1_ref[...], rb1_ref[...], W2r_ref[...],
                   rb2_ref[...])
        m_rt = msg(h_rt, rW1_ref[...], rb1_ref[...], W2r_ref[...],
                   rb2_ref[...])
        m_agg = ((m_up + m_dn) + m_lf) + m_rt

        xe = jnp.concatenate([self_bf, m_agg.astype(_BF)], axis=1)
        g = jnp.maximum(_dot(xe, eW1_ref[...]) + eb1_ref[...],
                        0.0).astype(_BF)
        hn = _dot(g, eW2_ref[...]) + eb2_ref[...]
        write_h(hn)

        @pl.when(s == 3)
        def _readout():
            hb = hn.astype(_BF).astype(jnp.float32)
            t = jnp.sum(hb * ro_row_ref[...], axis=1, keepdims=True)
            q = (t + robc_ref[...]).astype(_BF).astype(jnp.float32)
            part = jnp.sum(q * ro2_ref[...])

            @pl.when(rb == 0)
            def _():
                out_ref[0] = rdc_ref[...] + part

            @pl.when(rb > 0)
            def _():
                out_ref[0] = out_ref[0] + part


def _pad_rows(x2d):
    b = x2d.shape[0]
    x3 = x2d.reshape(b, _NR, _NB)
    x3 = jnp.pad(x3, ((0, 0), (0, 0), (0, _RW - _NB)))
    return x3.reshape(b, _N)


def _bfr(w):
    return w.astype(_BF)


def kernel(state, actor, up, down, left, right, in_W, in_b,
           row_W1, row_b1, row_W2, row_b2,
           col_W1, col_b1, col_W2, col_b2,
           emb_W1, emb_b1, emb_W2, emb_b2,
           ro_W, ro_b, ro2_W, ro2_b):
    B = state.shape[0]

    def _bdo(x, W):
        return jnp.dot(x.astype(_BF), W.astype(_BF),
                       precision=lax.Precision.HIGHEST,
                       preferred_element_type=jnp.float32)

    sa = jnp.concatenate([state.reshape(B, -1)[..., None],
                          actor.reshape(B, -1)[..., None]], axis=-1)
    h0 = _bdo(sa, in_W) + in_b
    h0p = jnp.pad(h0.reshape(B, _NR, _NB, _E),
                  ((0, 0), (0, 0), (0, _RW - _NB), (0, 0))).reshape(B, _N, _E)

    grid = (B, 4, _NRB)

    def full(shape):
        return pl.BlockSpec(shape, lambda b, s, r: (0,) * len(shape))

    out = pl.pallas_call(
        _gnn_body,
        grid=grid,
        in_specs=[
            pl.BlockSpec((1, _RN, _E), lambda b, s, r: (b, r, 0)),
            full((2, _E)),
            full((1, _E)),
            full((2 * _E, _U)),
            full((2 * _E, _U)),
            full((1, _U)),
            full((1, _U)),
            full((_U, _E)),
            full((_U, _E)),
            full((1, _E)),
            full((1, _E)),
            full((2 * _E, _U)),
            full((1, _U)),
            full((_U, _E)),
            full((1, _E)),
            full((1, _E)),
            full((1, 1)),
            pl.BlockSpec((_RN, 1), lambda b, s, r: (r, 0)),
            full((1, 1)),
        ],
        out_specs=pl.BlockSpec((1, 1, 1), lambda b, s, r: (b, 0, 0)),
        out_shape=jax.ShapeDtypeStruct((B, 1, 1), jnp.float32),
        scratch_shapes=[pltpu.VMEM((2, _NP, _E), jnp.float32)],
        compiler_params=pltpu.CompilerParams(
            dimension_semantics=("arbitrary", "arbitrary", "arbitrary")),
    )(h0p,
      _bfr(in_W).astype(jnp.float32),
      in_b.reshape(1, _E),
      _bfr(col_W1), _bfr(row_W1),
      col_b1.reshape(1, _U), row_b1.reshape(1, _U),
      _bfr(col_W2), _bfr(row_W2),
      col_b2.reshape(1, _E), row_b2.reshape(1, _E),
      _bfr(emb_W1),
      emb_b1.reshape(1, _U),
      _bfr(emb_W2), emb_b2.reshape(1, _E),
      _bfr(ro_W).astype(jnp.float32).reshape(1, _E),
      ro_b.reshape(1, 1),
      _bfr(_pad_rows(ro2_W.reshape(1, -1))).astype(jnp.float32).reshape(_N, 1),
      ro2_b.reshape(1, 1))
    return out.reshape(B, 1)


# speedup vs baseline: 1.2453x; 1.0709x over previous
import jax
import jax.numpy as jnp
from jax import lax
from jax.experimental import pallas as pl
from jax.experimental.pallas import tpu as pltpu

_NB = 100
_RW = 104
_NR = 100
_N = _NR * _RW
_E = 128
_U = 256
_H = _RW
_GR = 50
_RN = _GR * _RW
_NRB = _NR // _GR
_NP = _N + 2 * _H

_BF = jnp.bfloat16


def _dot(a, b):
    return jnp.dot(a, b, preferred_element_type=jnp.float32)


def _rne(x):
    u = lax.bitcast_convert_type(x, jnp.uint32)
    r = (u + jnp.uint32(0x7FFF) + ((u >> 16) & jnp.uint32(1))) & jnp.uint32(0xFFFF0000)
    return lax.bitcast_convert_type(r, jnp.float32)


def _gnn_body(xsa_ref, in_W_ref, in_b_ref, cW1_ref, rW1_ref,
              cb1_ref, rb1_ref, W2c_ref, W2r_ref, cb2_ref,
              rb2_ref, eW1_ref, eb1_ref, eW2_ref, eb2_ref,
              ro_row_ref, robc_ref, ro2_ref, rdc_ref, out_ref, buf_ref):
    s = pl.program_id(1)
    rb = pl.program_id(2)
    base = rb * _RN
    p_in = (s + 1) % 2
    p_out = s % 2

    def write_h(h):
        buf_ref[p_out, pl.ds(base + _H, _RN), :] = h

        @pl.when(rb == 0)
        def _():
            buf_ref[p_out, pl.ds(0, _H), :] = h[0:_H]

        @pl.when(rb == _NRB - 1)
        def _():
            buf_ref[p_out, pl.ds(_N + _H, _H), :] = h[_RN - _H:_RN]

    @pl.when(s == 0)
    def _embed():
        h0 = xsa_ref[0]
        write_h(h0)

    @pl.when(s > 0)
    def _layer():
        S = buf_ref[p_in, pl.ds(base, _RN + 2 * _H), :].astype(_BF)
        self_bf = S[_H:_H + _RN]

        h_up = S[0:_RN]
        h_dn = S[2 * _H:2 * _H + _RN]
        i0 = lax.broadcasted_iota(jnp.int32, (_RN, 1), 0) % _RW
        h_lf = jnp.where(i0 == 0, self_bf, S[_H - 1:_H - 1 + _RN])
        h_rt = jnp.where(i0 == _NB - 1, self_bf, S[_H + 1:_H + 1 + _RN])

        def msg(nbr, W1, b1, W2, b2):
            x = jnp.concatenate([self_bf, nbr], axis=1)
            a = jnp.maximum(_dot(x, W1) + b1, 0.0).astype(_BF)
            return _dot(a, W2) + b2

        m_up = msg(h_up, cW1_ref[...], cb1_ref[...], W2c_ref[...],
                   cb2_ref[...])
        m_dn = msg(h_dn, cW1_ref[...], cb1_ref[...], W2c_ref[...],
                   cb2_ref[...])
        m_lf = msg(h_lf, rW1_ref[...], rb1_ref[...], W2r_ref[...],
                   rb2_ref[...])
        m_rt = msg(h_rt, rW1_ref[...], rb1_ref[...], W2r_ref[...],
                   rb2_ref[...])
        m_agg = ((m_up + m_dn) + m_lf) + m_rt

        xe = jnp.concatenate([self_bf, m_agg.astype(_BF)], axis=1)
        g = jnp.maximum(_dot(xe, eW1_ref[...]) + eb1_ref[...],
                        0.0).astype(_BF)
        hn = _dot(g, eW2_ref[...]) + eb2_ref[...]
        write_h(hn)

        @pl.when(s == 3)
        def _readout():
            hb = hn.astype(_BF).astype(jnp.float32)
            t = jnp.sum(hb * ro_row_ref[...], axis=1, keepdims=True)
            q = (t + robc_ref[...]).astype(_BF).astype(jnp.float32)
            part = jnp.sum(q * ro2_ref[...])

            @pl.when(rb == 0)
            def _():
                out_ref[0] = rdc_ref[...] + part

            @pl.when(rb > 0)
            def _():
                out_ref[0] = out_ref[0] + part


def _pad_rows(x2d):
    b = x2d.shape[0]
    x3 = x2d.reshape(b, _NR, _NB)
    x3 = jnp.pad(x3, ((0, 0), (0, 0), (0, _RW - _NB)))
    return x3.reshape(b, _N)


def _bfr(w):
    return w.astype(_BF)


def kernel(state, actor, up, down, left, right, in_W, in_b,
           row_W1, row_b1, row_W2, row_b2,
           col_W1, col_b1, col_W2, col_b2,
           emb_W1, emb_b1, emb_W2, emb_b2,
           ro_W, ro_b, ro2_W, ro2_b):
    B = state.shape[0]

    def _bdo(x, W):
        return jnp.dot(x.astype(_BF), W.astype(_BF),
                       precision=lax.Precision.HIGHEST,
                       preferred_element_type=jnp.float32)

    sa = jnp.concatenate([state.reshape(B, -1)[..., None],
                          actor.reshape(B, -1)[..., None]], axis=-1)
    h0 = _bdo(sa, in_W) + in_b
    h0p = jnp.pad(h0.reshape(B, _NR, _NB, _E),
                  ((0, 0), (0, 0), (0, _RW - _NB), (0, 0))).reshape(B, _N, _E)

    grid = (B, 4, _NRB)

    def full(shape):
        return pl.BlockSpec(shape, lambda b, s, r: (0,) * len(shape))

    out = pl.pallas_call(
        _gnn_body,
        grid=grid,
        in_specs=[
            pl.BlockSpec((1, _RN, _E), lambda b, s, r: (b, r, 0)),
            full((2, _E)),
            full((1, _E)),
            full((2 * _E, _U)),
            full((2 * _E, _U)),
            full((1, _U)),
            full((1, _U)),
            full((_U, _E)),
            full((_U, _E)),
            full((1, _E)),
            full((1, _E)),
            full((2 * _E, _U)),
            full((1, _U)),
            full((_U, _E)),
            full((1, _E)),
            full((1, _E)),
            full((1, 1)),
            pl.BlockSpec((_RN, 1), lambda b, s, r: (r, 0)),
            full((1, 1)),
        ],
        out_specs=pl.BlockSpec((1, 1, 1), lambda b, s, r: (b, 0, 0)),
        out_shape=jax.ShapeDtypeStruct((B, 1, 1), jnp.float32),
        scratch_shapes=[pltpu.VMEM((2, _NP, _E), jnp.float32)],
        compiler_params=pltpu.CompilerParams(
            dimension_semantics=("arbitrary", "arbitrary", "arbitrary")),
    )(h0p,
      _bfr(in_W).astype(jnp.float32),
      in_b.reshape(1, _E),
      _bfr(col_W1), _bfr(row_W1),
      col_b1.reshape(1, _U), row_b1.reshape(1, _U),
      _bfr(col_W2), _bfr(row_W2),
      col_b2.reshape(1, _E), row_b2.reshape(1, _E),
      _bfr(emb_W1),
      emb_b1.reshape(1, _U),
      _bfr(emb_W2), emb_b2.reshape(1, _E),
      _bfr(ro_W).astype(jnp.float32).reshape(1, _E),
      ro_b.reshape(1, 1),
      _bfr(_pad_rows(ro2_W.reshape(1, -1))).astype(jnp.float32).reshape(_N, 1),
      ro2_b.reshape(1, 1))
    return out.reshape(B, 1)


# speedup vs baseline: 1.3993x; 1.1237x over previous
import jax
import jax.numpy as jnp
from jax import lax
from jax.experimental import pallas as pl
from jax.experimental.pallas import tpu as pltpu

_NB = 100
_RW = 104
_NR = 100
_N = _NR * _RW
_E = 128
_U = 256
_H = _RW
_GR = 50
_RN = _GR * _RW
_NRB = _NR // _GR
_NP = _N + 2 * _H

_BF = jnp.bfloat16


def _dot(a, b):
    return jnp.dot(a, b, preferred_element_type=jnp.float32)


def _rne(x):
    u = lax.bitcast_convert_type(x, jnp.uint32)
    r = (u + jnp.uint32(0x7FFF) + ((u >> 16) & jnp.uint32(1))) & jnp.uint32(0xFFFF0000)
    return lax.bitcast_convert_type(r, jnp.float32)


def _gnn_body(xsa_ref, ci_ref, in_W_ref, in_b_ref, cW1_ref, rW1_ref,
              cb1_ref, rb1_ref, W2c_ref, W2r_ref, cb2_ref,
              rb2_ref, eW1_ref, eb1_ref, eW2_ref, eb2_ref,
              ro_row_ref, robc_ref, ro2_ref, rdc_ref, out_ref, buf_ref):
    s = pl.program_id(1)
    rb = pl.program_id(2)
    base = rb * _RN
    p_in = (s + 1) % 2
    p_out = s % 2

    def write_h(h):
        buf_ref[p_out, pl.ds(base + _H, _RN), :] = h

        @pl.when(rb == 0)
        def _():
            buf_ref[p_out, pl.ds(0, _H), :] = h[0:_H]

        @pl.when(rb == _NRB - 1)
        def _():
            buf_ref[p_out, pl.ds(_N + _H, _H), :] = h[_RN - _H:_RN]

    @pl.when(s == 0)
    def _embed():
        h0 = xsa_ref[0].astype(jnp.float32)
        write_h(h0)

    @pl.when(s > 0)
    def _layer():
        S = buf_ref[p_in, pl.ds(base, _RN + 2 * _H), :]
        self_ = S[_H:_H + _RN]

        h_up = S[0:_RN]
        h_dn = S[2 * _H:2 * _H + _RN]
        ci = ci_ref[...]
        h_lf = jnp.where(ci == 0, self_, S[_H - 1:_H - 1 + _RN])
        h_rt = jnp.where(ci == _NB - 1, self_, S[_H + 1:_H + 1 + _RN])

        def msg(nbr, W1, b1, W2, b2):
            x = jnp.concatenate([self_, nbr], axis=1).astype(_BF)
            a = jnp.maximum(_dot(x, W1) + b1, 0.0).astype(_BF)
            return _dot(a, W2) + b2

        m_up = msg(h_up, cW1_ref[...], cb1_ref[...], W2c_ref[...],
                   cb2_ref[...])
        m_dn = msg(h_dn, cW1_ref[...], cb1_ref[...], W2c_ref[...],
                   cb2_ref[...])
        m_lf = msg(h_lf, rW1_ref[...], rb1_ref[...], W2r_ref[...],
                   rb2_ref[...])
        m_rt = msg(h_rt, rW1_ref[...], rb1_ref[...], W2r_ref[...],
                   rb2_ref[...])
        m_agg = ((m_up + m_dn) + m_lf) + m_rt

        xe = jnp.concatenate([self_, m_agg], axis=1).astype(_BF)
        g = jnp.maximum(_dot(xe, eW1_ref[...]) + eb1_ref[...],
                        0.0).astype(_BF)
        hn = _dot(g, eW2_ref[...]) + eb2_ref[...]
        write_h(hn)

        @pl.when(s == 3)
        def _readout():
            hb = hn.astype(_BF).astype(jnp.float32)
            t = jnp.sum(hb * ro_row_ref[...], axis=1, keepdims=True)
            q = (t + robc_ref[...]).astype(_BF).astype(jnp.float32)
            part = jnp.sum(q * ro2_ref[...])

            @pl.when(rb == 0)
            def _():
                out_ref[0] = rdc_ref[...] + part

            @pl.when(rb > 0)
            def _():
                out_ref[0] = out_ref[0] + part


def _pad_rows(x2d):
    b = x2d.shape[0]
    x3 = x2d.reshape(b, _NR, _NB)
    x3 = jnp.pad(x3, ((0, 0), (0, 0), (0, _RW - _NB)))
    return x3.reshape(b, _N)


def _bfr(w):
    return w.astype(_BF)


def kernel(state, actor, up, down, left, right, in_W, in_b,
           row_W1, row_b1, row_W2, row_b2,
           col_W1, col_b1, col_W2, col_b2,
           emb_W1, emb_b1, emb_W2, emb_b2,
           ro_W, ro_b, ro2_W, ro2_b):
    B = state.shape[0]

    def _bdo(x, W):
        return jnp.dot(x.astype(_BF), W.astype(_BF),
                       precision=lax.Precision.HIGHEST,
                       preferred_element_type=jnp.float32)

    sa = jnp.concatenate([state.reshape(B, -1)[..., None],
                          actor.reshape(B, -1)[..., None]], axis=-1)
    h0 = _bdo(sa, in_W) + in_b
    h0p = jnp.pad(h0.reshape(B, _NR, _NB, _E),
                  ((0, 0), (0, 0), (0, _RW - _NB), (0, 0))).reshape(
                      B, _N, _E).astype(_BF)
    crow = jnp.concatenate([jnp.arange(_NB, dtype=jnp.int32),
                            jnp.full((_RW - _NB,), 50, jnp.int32)])
    ci = jnp.tile(crow, _NR).reshape(_N, 1)

    grid = (B, 4, _NRB)

    def full(shape):
        return pl.BlockSpec(shape, lambda b, s, r: (0,) * len(shape))

    out = pl.pallas_call(
        _gnn_body,
        grid=grid,
        in_specs=[
            pl.BlockSpec((1, _RN, _E), lambda b, s, r: (b, r, 0)),
            pl.BlockSpec((_RN, 1), lambda b, s, r: (r, 0)),
            full((2, _E)),
            full((1, _E)),
            full((2 * _E, _U)),
            full((2 * _E, _U)),
            full((1, _U)),
            full((1, _U)),
            full((_U, _E)),
            full((_U, _E)),
            full((1, _E)),
            full((1, _E)),
            full((2 * _E, _U)),
            full((1, _U)),
            full((_U, _E)),
            full((1, _E)),
            full((1, _E)),
            full((1, 1)),
            pl.BlockSpec((_RN, 1), lambda b, s, r: (r, 0)),
            full((1, 1)),
        ],
        out_specs=pl.BlockSpec((1, 1, 1), lambda b, s, r: (b, 0, 0)),
        out_shape=jax.ShapeDtypeStruct((B, 1, 1), jnp.float32),
        scratch_shapes=[pltpu.VMEM((2, _NP, _E), jnp.float32)],
        compiler_params=pltpu.CompilerParams(
            dimension_semantics=("arbitrary", "arbitrary", "arbitrary")),
    )(h0p, ci,
      _bfr(in_W).astype(jnp.float32),
      in_b.reshape(1, _E),
      _bfr(col_W1), _bfr(row_W1),
      col_b1.reshape(1, _U), row_b1.reshape(1, _U),
      _bfr(col_W2), _bfr(row_W2),
      col_b2.reshape(1, _E), row_b2.reshape(1, _E),
      _bfr(emb_W1),
      emb_b1.reshape(1, _U),
      _bfr(emb_W2), emb_b2.reshape(1, _E),
      _bfr(ro_W).astype(jnp.float32).reshape(1, _E),
      ro_b.reshape(1, 1),
      _bfr(_pad_rows(ro2_W.reshape(1, -1))).astype(jnp.float32).reshape(_N, 1),
      ro2_b.reshape(1, 1))
    return out.reshape(B, 1)
